# Initial kernel scaffold; baseline (speedup 1.0000x reference)
import jax
import jax.numpy as jnp
from jax.experimental import pallas as pl


def kernel(user_emb, item_emb, text_feats, image_feats, text_W, image_W, weight_text, weight_image, adj_indices, adj_values):
    raise NotImplementedError("write your pallas kernel here")


# speedup vs baseline: 4.7323x; 4.7323x over previous
import functools

import jax
import jax.numpy as jnp
from jax import lax
from jax.experimental import pallas as pl
from jax.experimental.pallas import tpu as pltpu
from jax.experimental.pallas import tpu_sc as plsc

_NU = 30000
_NI = 20000
_NN = _NU + _NI
_D = 64
_H = 32
_NE = 800000

_NTILES = 16
_ROWS_PT = 3128
_ACC_N = _ROWS_PT * _NTILES
_EROWS_PT = 392
_EPT = _EROWS_PT * 128
_EP = _EPT * _NTILES
_CH_ROWS = 4
_CHUNK = _CH_ROWS * 128
_NCH = _EROWS_PT // _CH_ROWS


def _fuse_body(tf_ref, imf_ref, ie_ref, tw_ref, iw_ref, w_ref, out_ref):
    t = jnp.dot(tf_ref[...], tw_ref[...], preferred_element_type=jnp.float32)
    t = jnp.maximum(t, 0.0)
    tn = jnp.sqrt(jnp.sum(t * t, axis=1, keepdims=True))
    t = t / jnp.maximum(tn, 1e-12)
    im = jnp.dot(imf_ref[...], iw_ref[...], preferred_element_type=jnp.float32)
    im = jnp.maximum(im, 0.0)
    imn = jnp.sqrt(jnp.sum(im * im, axis=1, keepdims=True))
    im = im / jnp.maximum(imn, 1e-12)
    out_ref[...] = ie_ref[...] + w_ref[0] * t + w_ref[1] * im


def _fuse(item_emb, text_feats, image_feats, text_W, image_W, wt, wi):
    blk = 2000
    grid = _NI // blk
    w = jnp.stack([wt, wi]).astype(jnp.float32)
    return pl.pallas_call(
        _fuse_body,
        grid=(grid,),
        in_specs=[
            pl.BlockSpec((blk, 384), lambda i: (i, 0)),
            pl.BlockSpec((blk, 512), lambda i: (i, 0)),
            pl.BlockSpec((blk, _D), lambda i: (i, 0)),
            pl.BlockSpec((384, _D), lambda i: (0, 0)),
            pl.BlockSpec((512, _D), lambda i: (0, 0)),
            pl.BlockSpec(memory_space=pltpu.SMEM),
        ],
        out_specs=pl.BlockSpec((blk, _D), lambda i: (i, 0)),
        out_shape=jax.ShapeDtypeStruct((_NI, _D), jnp.float32),
    )(text_feats, image_feats, item_emb, text_W, image_W, w)


def _prop_body(ego0, cols1d, rows1d, vals1d, out1, out2, out3,
               acc, colsv, valsv, rows8, g, zbuf, sem):
    cid = lax.axis_index("c")
    sid = lax.axis_index("s")

    zero16 = jnp.zeros((16,), jnp.float32)
    for r in range(136):
        for k in range(2):
            zbuf[r, pl.ds(k * 16, 16)] = zero16

    row0 = sid * _ROWS_PT

    def zero_acc():
        for j in range(23):
            pltpu.sync_copy(zbuf, acc.at[pl.ds(row0 + j * 136, 136)])

    zero_acc()
    plsc.subcore_barrier()

    coff16 = jnp.full((16,), cid * _ACC_N, jnp.int32)
    e0t = sid * _EPT

    srcs = (ego0, out1, out2)
    outs = (out1, out2, out3)
    for l in range(3):
        src = srcs[l]

        def chunk_body(c, _, src=src):
            off = e0t + c * _CHUNK
            pltpu.sync_copy(cols1d.at[pl.ds(off, _CHUNK)], colsv)
            pltpu.sync_copy(vals1d.at[pl.ds(off, _CHUNK)], valsv)
            for j in range(_CH_ROWS):
                pltpu.sync_copy(rows1d.at[pl.ds(off + j * 128, 128)], rows8[j])
            def shift_body(k, _):
                colsv[pl.ds(k * 16, 16)] = colsv[pl.ds(k * 16, 16)] + coff16
                return 0
            lax.fori_loop(0, _CHUNK // 16, shift_body, 0, unroll=8)
            descs = [
                pltpu.async_copy(src.at[colsv.at[pl.ds(j * 128, 128)]],
                                 g.at[pl.ds(j * 128, 128)], sem)
                for j in range(_CH_ROWS)
            ]
            for d in descs:
                d.wait()

            def scale_body(eb, _):
                v16 = valsv[pl.ds(eb * 16, 16)]
                for i in range(16):
                    vv = lax.broadcast(v16[i], (16,))
                    e = eb * 16 + i
                    g[e, pl.ds(0, 16)] = g[e, pl.ds(0, 16)] * vv
                    g[e, pl.ds(16, 16)] = g[e, pl.ds(16, 16)] * vv
                return 0

            lax.fori_loop(0, _CHUNK // 16, scale_body, 0)

            for j in range(_CH_ROWS):
                pltpu.sync_copy(g.at[pl.ds(j * 128, 128)],
                                acc.at[rows8[j]], add=True)
            return 0

        lax.fori_loop(0, _NCH, chunk_body, 0)
        plsc.subcore_barrier()
        pltpu.sync_copy(acc.at[pl.ds(row0, _ROWS_PT)],
                        outs[l].at[pl.ds(cid * _ACC_N + row0, _ROWS_PT)])
        if l < 2:
            zero_acc()
        plsc.subcore_barrier()


def _prop(ego_f, cols2d, rows2d, vals2d):
    mesh = plsc.VectorSubcoreMesh(core_axis_name="c", subcore_axis_name="s")
    f = functools.partial(
        pl.kernel,
        out_type=(
            jax.ShapeDtypeStruct((2 * _ACC_N, _H), jnp.float32),
            jax.ShapeDtypeStruct((2 * _ACC_N, _H), jnp.float32),
            jax.ShapeDtypeStruct((2 * _ACC_N, _H), jnp.float32),
        ),
        mesh=mesh,
        compiler_params=pltpu.CompilerParams(use_tc_tiling_on_sc=False),
        scratch_types=[
            pltpu.VMEM_SHARED((_ACC_N, _H), jnp.float32),
            pltpu.VMEM((_CHUNK,), jnp.int32),
            pltpu.VMEM((_CHUNK,), jnp.float32),
            [pltpu.VMEM((128,), jnp.int32)] * _CH_ROWS,
            pltpu.VMEM((_CHUNK, _H), jnp.float32),
            pltpu.VMEM((136, _H), jnp.float32),
            pltpu.SemaphoreType.DMA,
        ],
    )(_prop_body)
    return f(ego_f, cols2d, rows2d, vals2d)


def _mean_body(a_ref, b_ref, c_ref, d_ref, out_ref):
    out_ref[...] = 0.25 * (a_ref[...] + b_ref[...] + c_ref[...] + d_ref[...])


def _mean(e0, e1, e2, e3):
    blk = 6256
    grid = (2 * _ACC_N) // blk
    spec = pl.BlockSpec((blk, _H), lambda i: (i, 0))
    return pl.pallas_call(
        _mean_body,
        grid=(grid,),
        in_specs=[spec] * 4,
        out_specs=spec,
        out_shape=jax.ShapeDtypeStruct((2 * _ACC_N, _H), jnp.float32),
    )(e0, e1, e2, e3)


def kernel(user_emb, item_emb, text_feats, image_feats, text_W, image_W,
           weight_text, weight_image, adj_indices, adj_values):
    fused = _fuse(item_emb, text_feats, image_feats, text_W, image_W,
                  weight_text, weight_image)
    ego = jnp.concatenate([user_emb, fused], axis=0)
    zpad = jnp.zeros((_ACC_N - _NN, _H), jnp.float32)
    ego_f = jnp.concatenate([ego[:, :_H], zpad, ego[:, _H:], zpad], axis=0)

    rows = adj_indices[0].astype(jnp.int32)
    cols = adj_indices[1].astype(jnp.int32)
    vals = adj_values.astype(jnp.float32)
    pad = _EP - _NE
    rows1d = jnp.concatenate([rows, jnp.zeros((pad,), jnp.int32)])
    cols1d = jnp.concatenate([cols, jnp.zeros((pad,), jnp.int32)])
    vals1d = jnp.concatenate([vals, jnp.zeros((pad,), jnp.float32)])

    e1, e2, e3 = _prop(ego_f, cols1d, rows1d, vals1d)
    mean_f = _mean(ego_f, e1, e2, e3)
    final = jnp.concatenate([mean_f[:_NN], mean_f[_ACC_N:_ACC_N + _NN]],
                            axis=1)
    return final[:_NU], final[_NU:]


# speedup vs baseline: 8.3538x; 1.7653x over previous
import functools

import jax
import jax.numpy as jnp
from jax import lax
from jax.experimental import pallas as pl
from jax.experimental.pallas import tpu as pltpu
from jax.experimental.pallas import tpu_sc as plsc

_NU = 30000
_NI = 20000
_NN = _NU + _NI
_D = 64
_H = 32
_NE = 800000

_NTILES = 16
_ROWS_PT = 3128
_ACC_N = _ROWS_PT * _NTILES
_HC = 256
_NSLOT = 196
_EPT = _HC * (_NSLOT + 4)
_EP = _EPT * _NTILES


def _fuse_body(tf_ref, imf_ref, ie_ref, tw_ref, iw_ref, w_ref, out_ref):
    t = jnp.dot(tf_ref[...], tw_ref[...], preferred_element_type=jnp.float32)
    t = jnp.maximum(t, 0.0)
    tn = jnp.sqrt(jnp.sum(t * t, axis=1, keepdims=True))
    t = t / jnp.maximum(tn, 1e-12)
    im = jnp.dot(imf_ref[...], iw_ref[...], preferred_element_type=jnp.float32)
    im = jnp.maximum(im, 0.0)
    imn = jnp.sqrt(jnp.sum(im * im, axis=1, keepdims=True))
    im = im / jnp.maximum(imn, 1e-12)
    out_ref[...] = ie_ref[...] + w_ref[0] * t + w_ref[1] * im


def _fuse(item_emb, text_feats, image_feats, text_W, image_W, wt, wi):
    blk = 2000
    grid = _NI // blk
    w = jnp.stack([wt, wi]).astype(jnp.float32)
    return pl.pallas_call(
        _fuse_body,
        grid=(grid,),
        in_specs=[
            pl.BlockSpec((blk, 384), lambda i: (i, 0)),
            pl.BlockSpec((blk, 512), lambda i: (i, 0)),
            pl.BlockSpec((blk, _D), lambda i: (i, 0)),
            pl.BlockSpec((384, _D), lambda i: (0, 0)),
            pl.BlockSpec((512, _D), lambda i: (0, 0)),
            pl.BlockSpec(memory_space=pltpu.SMEM),
        ],
        out_specs=pl.BlockSpec((blk, _D), lambda i: (i, 0)),
        out_shape=jax.ShapeDtypeStruct((_NI, _D), jnp.float32),
    )(text_feats, image_feats, item_emb, text_W, image_W, w)


def _prop_body(ego0, cols1d, rows1d, vals1d, out1, out2, out3,
               acc, colsv, valsv, rowsv, g, zbuf, lsem, gsem):
    cid = lax.axis_index("c")
    sid = lax.axis_index("s")

    zero16 = jnp.zeros((16,), jnp.float32)
    for r in range(136):
        for k in range(2):
            zbuf[r, pl.ds(k * 16, 16)] = zero16

    row0 = sid * _ROWS_PT

    def zero_acc():
        for j in range(23):
            pltpu.sync_copy(zbuf, acc.at[pl.ds(row0 + j * 136, 136)])

    zero_acc()
    plsc.subcore_barrier()

    coff16 = jnp.full((16,), cid * _ACC_N, jnp.int32)
    e0t = sid * _EPT

    def lin_fire(h, b):
        off = e0t + h * _HC
        pltpu.async_copy(cols1d.at[pl.ds(off, _HC)], colsv[b], lsem[b])
        pltpu.async_copy(vals1d.at[pl.ds(off, _HC)], valsv[b], lsem[b])
        pltpu.async_copy(rows1d.at[pl.ds(off, 128)], rowsv[2 * b], lsem[b])
        pltpu.async_copy(rows1d.at[pl.ds(off + 128, 128)], rowsv[2 * b + 1],
                         lsem[b])

    def lin_drain(b):
        off = e0t
        pltpu.make_async_copy(cols1d.at[pl.ds(off, _HC)], colsv[b],
                              lsem[b]).wait()
        pltpu.make_async_copy(vals1d.at[pl.ds(off, _HC)], valsv[b],
                              lsem[b]).wait()
        pltpu.make_async_copy(rows1d.at[pl.ds(off, 128)], rowsv[2 * b],
                              lsem[b]).wait()
        pltpu.make_async_copy(rows1d.at[pl.ds(off, 128)], rowsv[2 * b + 1],
                              lsem[b]).wait()

    def gfire(b, gb, src):
        lin_drain(b)

        def shift_body(k, _):
            colsv[b][pl.ds(k * 16, 16)] = colsv[b][pl.ds(k * 16, 16)] + coff16
            return 0

        lax.fori_loop(0, _HC // 16, shift_body, 0, unroll=8)
        for j in range(2):
            pltpu.async_copy(src.at[colsv[b].at[pl.ds(j * 128, 128)]],
                             g[gb].at[pl.ds(j * 128, 128)], gsem[gb])

    def consume(b, gb, src):
        for j in range(2):
            pltpu.make_async_copy(src.at[colsv[b].at[pl.ds(j * 128, 128)]],
                                  g[gb].at[pl.ds(j * 128, 128)],
                                  gsem[gb]).wait()

        def scale_body(eb, _):
            v16 = valsv[b][pl.ds(eb * 16, 16)]
            for i in range(16):
                vv = lax.broadcast(v16[i], (16,))
                e = eb * 16 + i
                g[gb][e, pl.ds(0, 16)] = g[gb][e, pl.ds(0, 16)] * vv
                g[gb][e, pl.ds(16, 16)] = g[gb][e, pl.ds(16, 16)] * vv
            return 0

        lax.fori_loop(0, _HC // 16, scale_body, 0)
        for j in range(2):
            pltpu.sync_copy(g[gb].at[pl.ds(j * 128, 128)],
                            acc.at[rowsv[2 * b + j]], add=True)

    srcs = (ego0, out1, out2)
    outs = (out1, out2, out3)
    for l in range(3):
        src = srcs[l]
        for b in range(3):
            lin_fire(b, b)
        gfire(0, 0, src)
        lin_fire(3, 3)

        def loop_body(c, _, src=src):
            base = 4 * c + 1
            for i, b in enumerate((1, 2, 3, 0)):
                k = base + i
                gfire(b, (1 + i) % 2, src)
                consume((b - 1) % 4, i % 2, src)
                lin_fire(k + 3, (b - 1) % 4)
            return 0

        lax.fori_loop(0, (_NSLOT + 4) // 4 - 1, loop_body, 0)
        consume(0, 0, src)
        for b in (1, 2, 3):
            lin_drain(b)

        plsc.subcore_barrier()
        pltpu.sync_copy(acc.at[pl.ds(row0, _ROWS_PT)],
                        outs[l].at[pl.ds(cid * _ACC_N + row0, _ROWS_PT)])
        if l < 2:
            zero_acc()
        plsc.subcore_barrier()


def _prop(ego_f, cols2d, rows2d, vals2d):
    mesh = plsc.VectorSubcoreMesh(core_axis_name="c", subcore_axis_name="s")
    f = functools.partial(
        pl.kernel,
        out_type=(
            jax.ShapeDtypeStruct((2 * _ACC_N, _H), jnp.float32),
            jax.ShapeDtypeStruct((2 * _ACC_N, _H), jnp.float32),
            jax.ShapeDtypeStruct((2 * _ACC_N, _H), jnp.float32),
        ),
        mesh=mesh,
        compiler_params=pltpu.CompilerParams(use_tc_tiling_on_sc=False),
        scratch_types=[
            pltpu.VMEM_SHARED((_ACC_N, _H), jnp.float32),
            [pltpu.VMEM((_HC,), jnp.int32)] * 4,
            [pltpu.VMEM((_HC,), jnp.float32)] * 4,
            [pltpu.VMEM((128,), jnp.int32)] * 8,
            [pltpu.VMEM((_HC, _H), jnp.float32)] * 2,
            pltpu.VMEM((136, _H), jnp.float32),
            [pltpu.SemaphoreType.DMA] * 4,
            [pltpu.SemaphoreType.DMA] * 2,
        ],
    )(_prop_body)
    return f(ego_f, cols2d, rows2d, vals2d)


def _mean_body(a_ref, b_ref, c_ref, d_ref, out_ref):
    out_ref[...] = 0.25 * (a_ref[...] + b_ref[...] + c_ref[...] + d_ref[...])


def _mean(e0, e1, e2, e3):
    blk = 6256
    grid = (2 * _ACC_N) // blk
    spec = pl.BlockSpec((blk, _H), lambda i: (i, 0))
    return pl.pallas_call(
        _mean_body,
        grid=(grid,),
        in_specs=[spec] * 4,
        out_specs=spec,
        out_shape=jax.ShapeDtypeStruct((2 * _ACC_N, _H), jnp.float32),
    )(e0, e1, e2, e3)


def kernel(user_emb, item_emb, text_feats, image_feats, text_W, image_W,
           weight_text, weight_image, adj_indices, adj_values):
    fused = _fuse(item_emb, text_feats, image_feats, text_W, image_W,
                  weight_text, weight_image)
    ego = jnp.concatenate([user_emb, fused], axis=0)
    zpad = jnp.zeros((_ACC_N - _NN, _H), jnp.float32)
    ego_f = jnp.concatenate([ego[:, :_H], zpad, ego[:, _H:], zpad], axis=0)

    rows = adj_indices[0].astype(jnp.int32)
    cols = adj_indices[1].astype(jnp.int32)
    vals = adj_values.astype(jnp.float32)
    real_pt = _NSLOT * _HC
    pad = _NTILES * real_pt - _NE

    def lay(x):
        x = jnp.concatenate([x, jnp.zeros((pad,), x.dtype)])
        x = x.reshape(_NTILES, real_pt)
        x = jnp.pad(x, ((0, 0), (0, _EPT - real_pt)))
        return x.reshape(-1)

    rows1d = lay(rows)
    cols1d = lay(cols)
    vals1d = lay(vals)

    e1, e2, e3 = _prop(ego_f, cols1d, rows1d, vals1d)
    mean_f = _mean(ego_f, e1, e2, e3)
    final = jnp.concatenate([mean_f[:_NN], mean_f[_ACC_N:_ACC_N + _NN]],
                            axis=1)
    return final[:_NU], final[_NU:]
